# Initial kernel scaffold; baseline (speedup 1.0000x reference)
import jax
import jax.numpy as jnp
from jax.experimental import pallas as pl


def kernel(hidden_states, proto):
    raise NotImplementedError("write your pallas kernel here")


# speedup vs baseline: 1.2814x; 1.2814x over previous
import functools

import jax
import jax.numpy as jnp
from jax.experimental import pallas as pl
from jax.experimental.pallas import tpu as pltpu

_NUM_EXPERTS = 64
_EPAD = 128
_HIDDEN = 2048
_TOP_K = 8
_TOKENS = 16384
_BT = 512

_NEG_INF = float("-inf")


def _proto_norm_block(p_ref, pn_ref):
    p = p_ref[...]
    pnorm = jnp.sqrt(jnp.sum(p * p, axis=1, keepdims=True))
    pn_ref[...] = (p / jnp.maximum(pnorm, 1e-12)).astype(jnp.bfloat16)


def _router_block(h_ref, pn_ref, w_ref, i_ref):
    h = h_ref[...]
    hnorm = jnp.sqrt(jnp.sum(h * h, axis=1, keepdims=True))
    hn = (h / jnp.maximum(hnorm, 1e-12)).astype(jnp.bfloat16)

    logits = jax.lax.dot_general(
        hn, pn_ref[...],
        (((1,), (1,)), ((), ())),
        preferred_element_type=jnp.float32,
    )
    iota = jax.lax.broadcasted_iota(jnp.int32, (_BT, _EPAD), 1)
    logits = jnp.where(iota < _NUM_EXPERTS, logits, _NEG_INF)

    m = jnp.max(logits, axis=1, keepdims=True)
    e = jnp.exp(logits - m)
    z = jnp.sum(e, axis=1, keepdims=True)
    probs = e / z

    cur = probs
    vals, ids = [], []
    for _ in range(_TOP_K):
        mx = jnp.max(cur, axis=1, keepdims=True)
        hit = cur == mx
        am = jnp.min(jnp.where(hit, iota, _EPAD), axis=1, keepdims=True)
        vals.append(mx)
        ids.append(am)
        cur = jnp.where(iota == am, -1.0, cur)

    w_ref[...] = jnp.concatenate(vals, axis=1)
    i_ref[...] = jnp.concatenate(ids, axis=1)


@jax.jit
def kernel(hidden_states, proto):
    proto_padded = jnp.pad(proto, ((0, _EPAD - _NUM_EXPERTS), (0, 0)))
    pn = pl.pallas_call(
        _proto_norm_block,
        out_shape=jax.ShapeDtypeStruct((_EPAD, _HIDDEN), jnp.bfloat16),
    )(proto_padded)

    grid = _TOKENS // _BT
    return pl.pallas_call(
        _router_block,
        grid=(grid,),
        in_specs=[
            pl.BlockSpec((_BT, _HIDDEN), lambda i: (i, 0)),
            pl.BlockSpec((_EPAD, _HIDDEN), lambda i: (0, 0)),
        ],
        out_specs=[
            pl.BlockSpec((_BT, _TOP_K), lambda i: (i, 0)),
            pl.BlockSpec((_BT, _TOP_K), lambda i: (i, 0)),
        ],
        out_shape=[
            jax.ShapeDtypeStruct((_TOKENS, _TOP_K), jnp.float32),
            jax.ShapeDtypeStruct((_TOKENS, _TOP_K), jnp.int32),
        ],
        compiler_params=pltpu.CompilerParams(
            dimension_semantics=("parallel",),
        ),
    )(hidden_states, pn)


# speedup vs baseline: 1.6529x; 1.2899x over previous
import functools

import jax
import jax.numpy as jnp
from jax.experimental import pallas as pl
from jax.experimental.pallas import tpu as pltpu

_NUM_EXPERTS = 64
_EPAD = 128
_HIDDEN = 2048
_TOP_K = 8
_TOKENS = 16384
_BT = 512

_NEG_INF = float("-inf")


def _proto_norm_block(p_ref, pn_ref):
    p = p_ref[...]
    pnorm = jnp.sqrt(jnp.sum(p * p, axis=1, keepdims=True))
    pn_ref[...] = (p / jnp.maximum(pnorm, 1e-12)).astype(jnp.bfloat16)


def _router_block(h_ref, pn_ref, w_ref, i_ref):
    h = h_ref[...]
    hnorm = jnp.sqrt(jnp.sum(h * h, axis=1, keepdims=True))
    hn = (h / jnp.maximum(hnorm, 1e-12)).astype(jnp.bfloat16)

    logits = jax.lax.dot_general(
        hn, pn_ref[...],
        (((1,), (1,)), ((), ())),
        preferred_element_type=jnp.float32,
    )
    iota_f = jax.lax.broadcasted_iota(
        jnp.int32, (_BT, _EPAD), 1).astype(jnp.float32)
    logits = jnp.where(iota_f < _NUM_EXPERTS, logits, _NEG_INF)

    e = jnp.exp(logits - 1.0)
    z = jnp.sum(e, axis=1, keepdims=True)
    probs = e * (1.0 / z)

    cur = probs
    vals, ids = [], []
    for _ in range(_TOP_K):
        mx = jnp.max(cur, axis=1, keepdims=True)
        hit = cur == mx
        am = jnp.min(jnp.where(hit, iota_f, float(_EPAD)), axis=1, keepdims=True)
        vals.append(mx)
        ids.append(am)
        cur = jnp.where(iota_f == am, -1.0, cur)

    w_ref[...] = jnp.concatenate(vals, axis=1)
    i_ref[...] = jnp.concatenate(ids, axis=1).astype(jnp.int32)


@jax.jit
def kernel(hidden_states, proto):
    proto_padded = jnp.pad(proto, ((0, _EPAD - _NUM_EXPERTS), (0, 0)))
    pn = pl.pallas_call(
        _proto_norm_block,
        out_shape=jax.ShapeDtypeStruct((_EPAD, _HIDDEN), jnp.bfloat16),
    )(proto_padded)

    grid = _TOKENS // _BT
    return pl.pallas_call(
        _router_block,
        grid=(grid,),
        in_specs=[
            pl.BlockSpec((_BT, _HIDDEN), lambda i: (i, 0)),
            pl.BlockSpec((_EPAD, _HIDDEN), lambda i: (0, 0)),
        ],
        out_specs=[
            pl.BlockSpec((_BT, _TOP_K), lambda i: (i, 0)),
            pl.BlockSpec((_BT, _TOP_K), lambda i: (i, 0)),
        ],
        out_shape=[
            jax.ShapeDtypeStruct((_TOKENS, _TOP_K), jnp.float32),
            jax.ShapeDtypeStruct((_TOKENS, _TOP_K), jnp.int32),
        ],
        compiler_params=pltpu.CompilerParams(
            dimension_semantics=("parallel",),
        ),
    )(hidden_states, pn)
